# baseline (speedup 1.0000x reference)
import functools

import jax
import jax.numpy as jnp
from jax import lax
from jax.experimental import pallas as pl
from jax.experimental.pallas import tpu as pltpu
from jax.experimental.pallas import tpu_sc as plsc

BS = 8
KP1 = 11
H = 64
W = 64
C = 64
NPX = BS * KP1 * H * W
NWORKERS = 32
PX_PER_TILE = NPX // NWORKERS
CHUNK = 128
NCHUNK = PX_PER_TILE // CHUNK
NBUF = 2
PW = W + 1
TROWS = 4352


def _sc_deform(table, gx, gy):
    mesh = plsc.VectorSubcoreMesh(core_axis_name="c", subcore_axis_name="s")

    @functools.partial(
        pl.kernel,
        out_type=jax.ShapeDtypeStruct((NPX, C), jnp.float32),
        mesh=mesh,
        compiler_params=pltpu.CompilerParams(needs_layout_passes=False),
        scratch_types=[
            pltpu.VMEM((PX_PER_TILE,), jnp.float32),
            pltpu.VMEM((PX_PER_TILE,), jnp.float32),
            [pltpu.VMEM((CHUNK,), jnp.int32)] * NBUF,
            [pltpu.VMEM((4 * CHUNK,), jnp.float32)] * NBUF,
            [pltpu.VMEM((CHUNK, 2 * C), jnp.int32)] * NBUF,
            [pltpu.VMEM((CHUNK, C), jnp.float32)] * NBUF,
            [pltpu.SemaphoreType.DMA] * NBUF,
            [pltpu.SemaphoreType.DMA] * NBUF,
            pltpu.VMEM_SHARED((TROWS, 2 * C), jnp.int32),
        ],
    )
    def deform_kernel(table_hbm, gx_hbm, gy_hbm, out_hbm,
                      gxv, gyv, idxs, ws, ps, outs, sems, osems, tabs):
        sid = lax.axis_index("s")
        wid = sid * 2 + lax.axis_index("c")
        base = wid * PX_PER_TILE
        trows_per = TROWS // 16
        pltpu.sync_copy(
            table_hbm.at[pl.ds(sid * trows_per, trows_per)],
            tabs.at[pl.ds(sid * trows_per, trows_per)])
        plsc.subcore_barrier()
        pltpu.sync_copy(gx_hbm.at[pl.ds(base, PX_PER_TILE)], gxv)
        pltpu.sync_copy(gy_hbm.at[pl.ds(base, PX_PER_TILE)], gyv)

        def compute_chunk(g, slot):
            idxr = idxs[slot]
            wr = ws[slot]
            for v in range(CHUNK // 16):
                s = pl.ds(g * CHUNK + v * 16, 16)
                x = (gxv[s] + 1.0) * (W / 2.0) - 0.5
                y = (gyv[s] + 1.0) * (H / 2.0) - 0.5
                xi = (x + 1.0).astype(jnp.int32) - 1
                yi = (y + 1.0).astype(jnp.int32) - 1
                wf = x - xi.astype(jnp.float32)
                nf = y - yi.astype(jnp.float32)
                ef = 1.0 - wf
                sf = 1.0 - nf
                idxr[pl.ds(v * 16, 16)] = (yi + 1) * PW + (xi + 1)
                wr[pl.ds(v * 16, 16)] = sf * ef
                wr[pl.ds(CHUNK + v * 16, 16)] = sf * wf
                wr[pl.ds(2 * CHUNK + v * 16, 16)] = nf * ef
                wr[pl.ds(3 * CHUNK + v * 16, 16)] = nf * wf

        def gather_copy(slot):
            return pltpu.make_async_copy(
                tabs.at[idxs[slot]], ps[slot], sems[slot])

        def gather_copy_hbm(slot):
            return pltpu.make_async_copy(
                table_hbm.at[idxs[slot]], ps[slot], sems[slot])

        def start_gather(g, slot):
            use_hbm = (g & 3) == 3

            @pl.when(use_hbm)
            def _():
                gather_copy_hbm(slot).start()

            @pl.when(jnp.logical_not(use_hbm))
            def _():
                gather_copy(slot).start()

        def out_copy(g, slot):
            return pltpu.make_async_copy(
                outs[slot], out_hbm.at[pl.ds(base + g * CHUNK, CHUNK)],
                osems[slot])

        def combine(g, slot):
            wr = ws[slot]
            pr = ps[slot]
            outv = outs[slot]

            @pl.when(g >= NBUF)
            def _():
                out_copy(g - NBUF, slot).wait()

            def body(q, carry):
                wnwv = wr[pl.ds(q * 16, 16)]
                wnev = wr[pl.ds(CHUNK + q * 16, 16)]
                wswv = wr[pl.ds(2 * CHUNK + q * 16, 16)]
                wsev = wr[pl.ds(3 * CHUNK + q * 16, 16)]
                fmt = plsc.PackFormat.INTERLEAVED
                for j in range(16):
                    ip = jnp.full((16,), j, jnp.int32)
                    wnw = wnwv.at[ip].get(mode="promise_in_bounds")
                    wne = wnev.at[ip].get(mode="promise_in_bounds")
                    wsw = wswv.at[ip].get(mode="promise_in_bounds")
                    wse = wsev.at[ip].get(mode="promise_in_bounds")
                    wnw2 = plsc.pack(wnw, wnw, format=fmt)
                    wne2 = plsc.pack(wne, wne, format=fmt)
                    wsw2 = plsc.pack(wsw, wsw, format=fmt)
                    wse2 = plsc.pack(wse, wse, format=fmt)
                    p = q * 16 + j
                    for t in range(C // 32):
                        def blk(c32):
                            return plsc.bitcast(
                                pr[p, pl.ds(c32 * 32 + t * 16, 16)],
                                jnp.bfloat16)

                        acc = (wnw2 * blk(0) + wne2 * blk(1)
                               + wsw2 * blk(2) + wse2 * blk(3))
                        acc_a, acc_b = plsc.unpack(acc, format=fmt)
                        outv[p, pl.ds(t * 32, 16)] = acc_a
                        outv[p, pl.ds(t * 32 + 16, 16)] = acc_b
                return carry

            lax.fori_loop(0, CHUNK // 16, body, 0)
            out_copy(g, slot).start()

        for g0 in range(NBUF - 1):
            compute_chunk(g0, g0)
            start_gather(g0, g0)

        def outer(i, carry):
            for b in range(NBUF):
                g = i * NBUF + b
                ahead = (b + NBUF - 1) % NBUF

                @pl.when(g + NBUF - 1 < NCHUNK)
                def _():
                    compute_chunk(g + NBUF - 1, ahead)
                    start_gather(g + NBUF - 1, ahead)

                gather_copy(b).wait()
                combine(g, b)
            return carry

        lax.fori_loop(0, NCHUNK // NBUF, outer, 0)
        for b in range(NBUF):
            out_copy(NCHUNK - NBUF + b, b).wait()

    return deform_kernel(table, gx, gy)


def kernel(source, sparse_motions):
    src = source[0]
    pad = jnp.pad(src, ((1, 1), (1, 1), (0, 0)))
    table = jnp.concatenate(
        [pad[:-1, :-1], pad[:-1, 1:], pad[1:, :-1], pad[1:, 1:]], axis=-1
    ).reshape(PW * PW, 4 * C)
    table = (table.reshape(PW * PW, 8, 2, 16).swapaxes(2, 3)
             .reshape(PW * PW, 2 * C, 2).astype(jnp.bfloat16))
    table = jax.lax.bitcast_convert_type(table, jnp.int32)
    table = jnp.concatenate(
        [table, jnp.zeros((TROWS - PW * PW, 2 * C), jnp.int32)], axis=0)
    grid = sparse_motions.reshape(NPX, 2)
    gx = grid[:, 0]
    gy = grid[:, 1]
    out = _sc_deform(table, gx, gy)
    return out.reshape(BS, KP1, H, W, C)


# speedup vs baseline: 1.0628x; 1.0628x over previous
import functools

import jax
import jax.numpy as jnp
from jax import lax
from jax.experimental import pallas as pl
from jax.experimental.pallas import tpu as pltpu
from jax.experimental.pallas import tpu_sc as plsc

BS = 8
KP1 = 11
H = 64
W = 64
C = 64
NPX = BS * KP1 * H * W
NWORKERS = 32
PX_PER_TILE = NPX // NWORKERS
CHUNK = 128
NCHUNK = PX_PER_TILE // CHUNK
NBUF = 2
PW = W + 1
TROWS = 4352


def _sc_deform(table, gx, gy):
    mesh = plsc.VectorSubcoreMesh(core_axis_name="c", subcore_axis_name="s")

    @functools.partial(
        pl.kernel,
        out_type=jax.ShapeDtypeStruct((NPX, C), jnp.float32),
        mesh=mesh,
        compiler_params=pltpu.CompilerParams(needs_layout_passes=False),
        scratch_types=[
            pltpu.VMEM((PX_PER_TILE,), jnp.float32),
            pltpu.VMEM((PX_PER_TILE,), jnp.float32),
            [pltpu.VMEM((CHUNK,), jnp.int32)] * NBUF,
            [pltpu.VMEM((4 * CHUNK,), jnp.float32)] * NBUF,
            [pltpu.VMEM((CHUNK, 2 * C), jnp.int32)] * NBUF,
            [pltpu.VMEM((CHUNK, C), jnp.float32)] * NBUF,
            [pltpu.SemaphoreType.DMA] * NBUF,
            [pltpu.SemaphoreType.DMA] * NBUF,
            pltpu.VMEM_SHARED((TROWS, 2 * C), jnp.int32),
        ],
    )
    def deform_kernel(table_hbm, gx_hbm, gy_hbm, out_hbm,
                      gxv, gyv, idxs, ws, ps, outs, sems, osems, tabs):
        sid = lax.axis_index("s")
        wid = sid * 2 + lax.axis_index("c")
        base = wid * PX_PER_TILE
        trows_per = TROWS // 16
        pltpu.sync_copy(
            table_hbm.at[pl.ds(sid * trows_per, trows_per)],
            tabs.at[pl.ds(sid * trows_per, trows_per)])
        plsc.subcore_barrier()
        pltpu.sync_copy(gx_hbm.at[pl.ds(base, PX_PER_TILE)], gxv)
        pltpu.sync_copy(gy_hbm.at[pl.ds(base, PX_PER_TILE)], gyv)

        def compute_chunk(g, slot):
            idxr = idxs[slot]
            wr = ws[slot]
            for v in range(CHUNK // 16):
                s = pl.ds(g * CHUNK + v * 16, 16)
                x = (gxv[s] + 1.0) * (W / 2.0) - 0.5
                y = (gyv[s] + 1.0) * (H / 2.0) - 0.5
                xi = (x + 1.0).astype(jnp.int32) - 1
                yi = (y + 1.0).astype(jnp.int32) - 1
                wf = x - xi.astype(jnp.float32)
                nf = y - yi.astype(jnp.float32)
                ef = 1.0 - wf
                sf = 1.0 - nf
                idxr[pl.ds(v * 16, 16)] = (yi + 1) * PW + (xi + 1)
                wr[pl.ds(v * 16, 16)] = sf * ef
                wr[pl.ds(CHUNK + v * 16, 16)] = sf * wf
                wr[pl.ds(2 * CHUNK + v * 16, 16)] = nf * ef
                wr[pl.ds(3 * CHUNK + v * 16, 16)] = nf * wf

        def gather_copy(slot):
            return pltpu.make_async_copy(
                tabs.at[idxs[slot]], ps[slot], sems[slot])

        def out_copy(g, slot):
            return pltpu.make_async_copy(
                outs[slot], out_hbm.at[pl.ds(base + g * CHUNK, CHUNK)],
                osems[slot])

        def combine(g, slot):
            wr = ws[slot]
            pr = ps[slot]
            outv = outs[slot]

            @pl.when(g >= NBUF)
            def _():
                out_copy(g - NBUF, slot).wait()

            def body(q, carry):
                wnwv = wr[pl.ds(q * 16, 16)]
                wnev = wr[pl.ds(CHUNK + q * 16, 16)]
                wswv = wr[pl.ds(2 * CHUNK + q * 16, 16)]
                wsev = wr[pl.ds(3 * CHUNK + q * 16, 16)]
                fmt = plsc.PackFormat.INTERLEAVED
                for j in range(16):
                    ip = jnp.full((16,), j, jnp.int32)
                    wnw = wnwv.at[ip].get(mode="promise_in_bounds")
                    wne = wnev.at[ip].get(mode="promise_in_bounds")
                    wsw = wswv.at[ip].get(mode="promise_in_bounds")
                    wse = wsev.at[ip].get(mode="promise_in_bounds")
                    wnw2 = plsc.pack(wnw, wnw, format=fmt)
                    wne2 = plsc.pack(wne, wne, format=fmt)
                    wsw2 = plsc.pack(wsw, wsw, format=fmt)
                    wse2 = plsc.pack(wse, wse, format=fmt)
                    p = q * 16 + j
                    for t in range(C // 32):
                        def blk(c32):
                            return plsc.bitcast(
                                pr[p, pl.ds(c32 * 32 + t * 16, 16)],
                                jnp.bfloat16)

                        acc = (wnw2 * blk(0) + wne2 * blk(1)
                               + wsw2 * blk(2) + wse2 * blk(3))
                        acc_a, acc_b = plsc.unpack(acc, format=fmt)
                        outv[p, pl.ds(t * 32, 16)] = acc_a
                        outv[p, pl.ds(t * 32 + 16, 16)] = acc_b
                return carry

            lax.fori_loop(0, CHUNK // 16, body, 0)
            out_copy(g, slot).start()

        for g0 in range(NBUF - 1):
            compute_chunk(g0, g0)
            gather_copy(g0).start()

        def outer(i, carry):
            for b in range(NBUF):
                g = i * NBUF + b
                ahead = (b + NBUF - 1) % NBUF

                @pl.when(g + NBUF - 1 < NCHUNK)
                def _():
                    compute_chunk(g + NBUF - 1, ahead)
                    gather_copy(ahead).start()

                gather_copy(b).wait()
                combine(g, b)
            return carry

        lax.fori_loop(0, NCHUNK // NBUF, outer, 0)
        for b in range(NBUF):
            out_copy(NCHUNK - NBUF + b, b).wait()

    return deform_kernel(table, gx, gy)


def kernel(source, sparse_motions):
    src = source[0]
    pad = jnp.pad(src, ((1, 1), (1, 1), (0, 0)))
    table = jnp.concatenate(
        [pad[:-1, :-1], pad[:-1, 1:], pad[1:, :-1], pad[1:, 1:]], axis=-1
    ).reshape(PW * PW, 4 * C)
    table = (table.reshape(PW * PW, 8, 2, 16).swapaxes(2, 3)
             .reshape(PW * PW, 2 * C, 2).astype(jnp.bfloat16))
    table = jax.lax.bitcast_convert_type(table, jnp.int32)
    table = jnp.concatenate(
        [table, jnp.zeros((TROWS - PW * PW, 2 * C), jnp.int32)], axis=0)
    grid = sparse_motions.reshape(NPX, 2)
    gx = grid[:, 0]
    gy = grid[:, 1]
    out = _sc_deform(table, gx, gy)
    return out.reshape(BS, KP1, H, W, C)
